# baseline (speedup 1.0000x reference)
import jax
import jax.numpy as jnp
from jax.experimental import pallas as pl
from jax.experimental.pallas import tpu as pltpu

_BLK = 512
_NBUF = 4


def _router_body(x_hbm, w_ref, noise_ref, out_ref, xbuf, sems):
    n_steps = x_hbm.shape[0] // _BLK
    w_bf16 = w_ref[...].astype(jnp.bfloat16)

    def start(i):
        slot = i % _NBUF
        pltpu.make_async_copy(
            x_hbm.at[pl.ds(i * _BLK, _BLK), :],
            xbuf.at[slot],
            sems.at[slot],
        ).start()

    def wait(i):
        slot = i % _NBUF
        pltpu.make_async_copy(
            x_hbm.at[pl.ds(i * _BLK, _BLK), :],
            xbuf.at[slot],
            sems.at[slot],
        ).wait()

    for i in range(min(_NBUF, n_steps)):
        start(i)
    for i in range(n_steps):
        wait(i)
        slot = i % _NBUF
        acc = jax.lax.dot_general(
            xbuf[slot].astype(jnp.bfloat16),
            w_bf16,
            dimension_numbers=(((1,), (1,)), ((), ())),
            preferred_element_type=jnp.float32,
        )
        out_ref[pl.ds(i * _BLK, _BLK), :] = acc + noise_ref[pl.ds(i * _BLK, _BLK), :]
        if i + _NBUF < n_steps:
            start(i + _NBUF)


def kernel(x, W, noise):
    tokens, d_model = x.shape
    n_experts = W.shape[0]
    return pl.pallas_call(
        _router_body,
        in_specs=[
            pl.BlockSpec(memory_space=pltpu.MemorySpace.HBM),
            pl.BlockSpec(memory_space=pltpu.MemorySpace.VMEM),
            pl.BlockSpec(memory_space=pltpu.MemorySpace.VMEM),
        ],
        out_specs=pl.BlockSpec(memory_space=pltpu.MemorySpace.VMEM),
        out_shape=jax.ShapeDtypeStruct((tokens, n_experts), jnp.float32),
        scratch_shapes=[
            pltpu.VMEM((_NBUF, _BLK, d_model), jnp.float32),
            pltpu.SemaphoreType.DMA((_NBUF,)),
        ],
    )(x, W, noise)


# speedup vs baseline: 1.0641x; 1.0641x over previous
import jax
import jax.numpy as jnp
from jax.experimental import pallas as pl


def _router_block(x_ref, w_ref, noise_ref, out_ref):
    i = pl.program_id(0)
    blk = x_ref.shape[0]
    acc = jax.lax.dot_general(
        x_ref[...],
        w_ref[...],
        dimension_numbers=(((1,), (1,)), ((), ())),
        preferred_element_type=jnp.float32,
    )
    out_ref[pl.ds(i * blk, blk), :] = acc + noise_ref[pl.ds(i * blk, blk), :]


def kernel(x, W, noise):
    tokens, d_model = x.shape
    n_experts = W.shape[0]
    blk = 512
    return pl.pallas_call(
        _router_block,
        grid=(tokens // blk,),
        in_specs=[
            pl.BlockSpec((blk, d_model), lambda i: (i, 0)),
            pl.BlockSpec((n_experts, d_model), lambda i: (0, 0)),
            pl.BlockSpec((tokens, n_experts), lambda i: (0, 0)),
        ],
        out_specs=pl.BlockSpec((tokens, n_experts), lambda i: (0, 0)),
        out_shape=jax.ShapeDtypeStruct((tokens, n_experts), jnp.float32),
    )(x, W, noise)
